# Initial kernel scaffold; baseline (speedup 1.0000x reference)
import jax
import jax.numpy as jnp
from jax.experimental import pallas as pl


def kernel(node_ebd_ext, edge_ebd, h2, angle_ebd, nlist, nlist_mask, sw, a_nlist, a_nlist_mask, a_sw, W_self, b_self, W_sym, b_sym, W_ne, b_ne, W_es, b_es, res_n, res_e):
    raise NotImplementedError("write your pallas kernel here")


# speedup vs baseline: 1.6651x; 1.6651x over previous
import functools

import jax
import jax.numpy as jnp
from jax import lax
from jax.experimental import pallas as pl
from jax.experimental.pallas import tpu as pltpu
from jax.experimental.pallas import tpu_sc as plsc

_NLOC = 10000
_NNEI = 32
_NDIM = 128
_EDIM = 16
_AXIS = 4

_NW = 32
_ROWS_PER_W = (_NLOC * _NNEI) // _NW
_CHUNK = 80
_NCHUNK = _ROWS_PER_W // _CHUNK


def _sc_gather(table, idx3):
    mesh = plsc.VectorSubcoreMesh(core_axis_name="c", subcore_axis_name="s")

    def body(table_hbm, idx_hbm, out_hbm, idx_v, rows_v, gsem):
        wid = lax.axis_index("s") * 2 + lax.axis_index("c")
        base = wid * _ROWS_PER_W
        pltpu.sync_copy(idx_hbm.at[wid], idx_v)

        def step(c, carry):
            pltpu.async_copy(table_hbm.at[idx_v.at[c]], rows_v, gsem).wait()
            pltpu.sync_copy(rows_v, out_hbm.at[pl.ds(base + c * _CHUNK, _CHUNK)])
            return carry

        lax.fori_loop(0, _NCHUNK, step, 0, unroll=False)

    f = pl.kernel(
        body,
        out_type=jax.ShapeDtypeStruct((_NLOC * _NNEI, _NDIM), jnp.float32),
        mesh=mesh,
        scratch_types=[
            pltpu.VMEM((_NCHUNK, _CHUNK), jnp.int32),
            pltpu.VMEM((_CHUNK, _NDIM), jnp.float32),
            pltpu.SemaphoreType.DMA,
        ],
    )
    return f(table, idx3)


def _silu(x):
    return x / (1.0 + jnp.exp(-x))


_BLK = 200


def _tc_body(node_ref, g_ref, e_ref, h2_ref, sw_ref,
             w_self_ref, b_self_ref,
             w_sn_ref, w_se_ref, b_sym_ref,
             w_ne_a_ref, w_ne_g_ref, w_ne_e_ref, b_ne_ref,
             w_es_a_ref, w_es_g_ref, w_es_e_ref, b_es_ref,
             res_n_ref, res_e_ref,
             nout_ref, eout_ref):
    node = node_ref[...]
    g = g_ref[...]
    e = e_ref[...]
    h2 = h2_ref[...]
    sw = sw_ref[...]

    f32 = jnp.float32
    dot = functools.partial(jax.lax.dot_general, preferred_element_type=f32)

    def mm(a, b):
        return dot(a, b, (((a.ndim - 1,), (0,)), ((), ())))

    node_self = _silu(mm(node, w_self_ref[...]) + b_self_ref[...])

    hg_e = []
    hg_n = []
    for t in range(3):
        w_t = (h2[:, :, t] * sw)[:, :, None]
        hg_e.append(jnp.sum(e * w_t, axis=1) * (1.0 / _NNEI))
        hg_n.append(jnp.sum(g * w_t, axis=1) * (1.0 / _NNEI))

    sym_pre = b_sym_ref[...]
    for a in range(_AXIS):
        ge_a = sum(hg_e[t] * hg_e[t][:, a:a + 1] for t in range(3)) * (1.0 / 3.0)
        gn_a = sum(hg_n[t] * hg_n[t][:, a:a + 1] for t in range(3)) * (1.0 / 3.0)
        sym_pre = sym_pre + mm(ge_a, w_se_ref[a]) + mm(gn_a, w_sn_ref[a])
    node_sym = _silu(sym_pre)

    blk = node.shape[0]
    g_flat = g.reshape(blk * _NNEI, _NDIM)
    e_flat = e.reshape(blk * _NNEI, _EDIM)

    a_ne = mm(node, w_ne_a_ref[...])
    t_ne = mm(g_flat, w_ne_g_ref[...]) + mm(e_flat, w_ne_e_ref[...])
    arg = a_ne[:, None, :] + t_ne.reshape(blk, _NNEI, _NDIM) + b_ne_ref[...]
    msg = _silu(arg) * sw[:, :, None]
    node_edge = jnp.sum(msg, axis=1) * (1.0 / _NNEI)

    a_es = mm(node, w_es_a_ref[...])
    t_es = mm(g_flat, w_es_g_ref[...]) + mm(e_flat, w_es_e_ref[...])
    earg = a_es[:, None, :] + t_es.reshape(blk, _NNEI, _EDIM) + b_es_ref[...]
    edge_self = _silu(earg)

    res_n = res_n_ref[...]
    res_e = res_e_ref[...]
    nout_ref[...] = (node + res_n[0] * node_self + res_n[1] * node_sym
                     + res_n[2] * node_edge)
    eout_ref[...] = e + res_e[0] * edge_self


def _tc_dense(node, g, e, h2, sw, w_self, b_self, w_sn, w_se, b_sym,
              w_ne_a, w_ne_g, w_ne_e, b_ne, w_es_a, w_es_g, w_es_e, b_es,
              res_n, res_e):
    grid = (_NLOC // _BLK,)

    def rows(*tail):
        return pl.BlockSpec((_BLK,) + tail, lambda i: (i,) + (0,) * len(tail))

    def whole(shape):
        return pl.BlockSpec(shape, lambda i: (0,) * len(shape))

    return pl.pallas_call(
        _tc_body,
        grid=grid,
        in_specs=[
            rows(_NDIM),
            rows(_NNEI, _NDIM),
            rows(_NNEI, _EDIM),
            rows(_NNEI, 3),
            rows(_NNEI),
            whole(w_self.shape), whole(b_self.shape),
            whole(w_sn.shape), whole(w_se.shape), whole(b_sym.shape),
            whole(w_ne_a.shape), whole(w_ne_g.shape), whole(w_ne_e.shape),
            whole(b_ne.shape),
            whole(w_es_a.shape), whole(w_es_g.shape), whole(w_es_e.shape),
            whole(b_es.shape),
            whole(res_n.shape), whole(res_e.shape),
        ],
        out_specs=[
            rows(_NDIM),
            rows(_NNEI, _EDIM),
        ],
        out_shape=[
            jax.ShapeDtypeStruct((_NLOC, _NDIM), jnp.float32),
            jax.ShapeDtypeStruct((_NLOC, _NNEI, _EDIM), jnp.float32),
        ],
    )(node, g, e, h2, sw, w_self, b_self, w_sn, w_se, b_sym,
      w_ne_a, w_ne_g, w_ne_e, b_ne, w_es_a, w_es_g, w_es_e, b_es,
      res_n, res_e)


def kernel(node_ebd_ext, edge_ebd, h2, angle_ebd, nlist, nlist_mask, sw,
           a_nlist, a_nlist_mask, a_sw,
           W_self, b_self, W_sym, b_sym, W_ne, b_ne, W_es, b_es, res_n, res_e):
    node = node_ebd_ext[0]
    e = edge_ebd[0]
    h2_ = h2[0]
    sw_ = sw[0]
    idx3 = nlist[0].astype(jnp.int32).reshape(_NW, _NCHUNK, _CHUNK)

    g_flat = _sc_gather(node, idx3)
    g = g_flat.reshape(_NLOC, _NNEI, _NDIM)

    w_ne_a = W_ne[:_NDIM]
    w_ne_g = W_ne[_NDIM:2 * _NDIM]
    w_ne_e = W_ne[2 * _NDIM:]
    w_es_a = W_es[:_NDIM]
    w_es_g = W_es[_NDIM:2 * _NDIM]
    w_es_e = W_es[2 * _NDIM:]
    n_e_sym = _EDIM * _AXIS
    w_se = jnp.stack([W_sym[a:n_e_sym:_AXIS] for a in range(_AXIS)])
    w_sn = jnp.stack([W_sym[n_e_sym + a::_AXIS] for a in range(_AXIS)])

    n_out, e_out = _tc_dense(
        node, g, e, h2_, sw_,
        W_self, b_self.reshape(1, -1), w_sn, w_se, b_sym.reshape(1, -1),
        w_ne_a, w_ne_g, w_ne_e, b_ne.reshape(1, -1),
        w_es_a, w_es_g, w_es_e, b_es.reshape(1, -1),
        res_n, res_e)

    return n_out[None], e_out[None], angle_ebd
